# baseline (speedup 1.0000x reference)
import jax
import jax.numpy as jnp
from jax.experimental import pallas as pl
from jax.experimental.pallas import tpu as pltpu

_N = 10000
_D = 128
_SIZES = (1000, 5000, 3000, 1000)
_OFFS = (0, 1000, 6000, 9000)


def _mlp_chunk(x, w1, w2):
    h = jax.lax.dot_general(
        x, w1, (((1,), (1,)), ((), ())),
        preferred_element_type=jnp.float32,
    )
    h = jnp.maximum(h, 0.0)
    return jax.lax.dot_general(
        h, w2, (((1,), (1,)), ((), ())),
        preferred_element_type=jnp.float32,
    )


def _streaming_kernel(x_hbm, w_emb_ref, w_out_ref, o_hbm, *scratch):
    nc = len(_SIZES)
    x_bufs = scratch[:nc]
    o_bufs = scratch[nc:2 * nc]
    in_sem, out_sem = scratch[2 * nc], scratch[2 * nc + 1]

    def in_copy(i):
        return pltpu.make_async_copy(
            x_hbm.at[pl.ds(_OFFS[i], _SIZES[i]), :], x_bufs[i], in_sem.at[i])

    def out_copy(i):
        return pltpu.make_async_copy(
            o_bufs[i], o_hbm.at[pl.ds(_OFFS[i], _SIZES[i]), :], out_sem.at[i])

    for i in range(nc):
        in_copy(i).start()
    w1 = w_emb_ref[...]
    w2 = w_out_ref[...]
    for i in range(nc):
        in_copy(i).wait()
        o_bufs[i][...] = _mlp_chunk(x_bufs[i][...], w1, w2)
        out_copy(i).start()
    for i in range(nc):
        out_copy(i).wait()


def kernel(features, edge_index, W_emb, W_out, betas):
    del edge_index, betas
    bufs = [pltpu.VMEM((s, _D), jnp.float32) for s in _SIZES]
    return pl.pallas_call(
        _streaming_kernel,
        in_specs=[
            pl.BlockSpec(memory_space=pltpu.MemorySpace.HBM),
            pl.BlockSpec(memory_space=pltpu.MemorySpace.VMEM),
            pl.BlockSpec(memory_space=pltpu.MemorySpace.VMEM),
        ],
        out_specs=pl.BlockSpec(memory_space=pltpu.MemorySpace.HBM),
        out_shape=jax.ShapeDtypeStruct((_N, _D), jnp.float32),
        scratch_shapes=bufs + bufs + [
            pltpu.SemaphoreType.DMA((len(_SIZES),)),
            pltpu.SemaphoreType.DMA((len(_SIZES),)),
        ],
    )(features, W_emb, W_out)


# speedup vs baseline: 1.1800x; 1.1800x over previous
import jax
import jax.numpy as jnp
from jax.experimental import pallas as pl
from jax.experimental.pallas import tpu as pltpu

_N = 10000
_D = 128
_SIZES = (1000, 4000, 4000, 1000)
_OFFS = (0, 1000, 5000, 9000)


def _mlp_chunk(x, w1, w2):
    h = jax.lax.dot_general(
        x, w1, (((1,), (1,)), ((), ())),
        preferred_element_type=jnp.float32,
    )
    h = jnp.maximum(h, 0.0)
    return jax.lax.dot_general(
        h, w2, (((1,), (1,)), ((), ())),
        preferred_element_type=jnp.float32,
    )


def _streaming_kernel(x_hbm, w_emb_ref, w_out_ref, o_hbm, *scratch):
    nc = len(_SIZES)
    x_bufs = scratch[:nc]
    o_bufs = scratch[nc:2 * nc]
    in_sem, out_sem = scratch[2 * nc], scratch[2 * nc + 1]

    def in_copy(i):
        return pltpu.make_async_copy(
            x_hbm.at[pl.ds(_OFFS[i], _SIZES[i]), :], x_bufs[i], in_sem.at[i])

    def out_copy(i):
        return pltpu.make_async_copy(
            o_bufs[i], o_hbm.at[pl.ds(_OFFS[i], _SIZES[i]), :], out_sem.at[i])

    for i in range(nc):
        in_copy(i).start()
    w1 = w_emb_ref[...]
    w2 = w_out_ref[...]
    for i in range(nc):
        in_copy(i).wait()
        o_bufs[i][...] = _mlp_chunk(x_bufs[i][...], w1, w2)
        out_copy(i).start()
    for i in range(nc):
        out_copy(i).wait()


def kernel(features, edge_index, W_emb, W_out, betas):
    del edge_index, betas
    bufs = [pltpu.VMEM((s, _D), jnp.float32) for s in _SIZES]
    return pl.pallas_call(
        _streaming_kernel,
        in_specs=[
            pl.BlockSpec(memory_space=pltpu.MemorySpace.HBM),
            pl.BlockSpec(memory_space=pltpu.MemorySpace.VMEM),
            pl.BlockSpec(memory_space=pltpu.MemorySpace.VMEM),
        ],
        out_specs=pl.BlockSpec(memory_space=pltpu.MemorySpace.HBM),
        out_shape=jax.ShapeDtypeStruct((_N, _D), jnp.float32),
        scratch_shapes=bufs + bufs + [
            pltpu.SemaphoreType.DMA((len(_SIZES),)),
            pltpu.SemaphoreType.DMA((len(_SIZES),)),
        ],
    )(features, W_emb, W_out)
